# baseline (device time: 261418 ns/iter reference)
import jax
import jax.numpy as jnp
from jax import lax
from jax.experimental import pallas as pl
from jax.experimental.pallas import tpu as pltpu

K = 32


def kernel(x):
    m, n = x.shape
    h = m // 2
    rp = h // K

    def body(x_hbm, out_hbm,
             f32s, f32o, y_send_buf, own_bf, y_recv_buf, x_recv_buf,
             f32s_sems, f32o_sems, st_own_sems,
             y_send_sems, y_recv_sems, x_send_sems, x_recv_sems,
             st_a_sems, st_y_sems, st_x_sems):
        my_x = lax.axis_index("x")
        my_y = lax.axis_index("y")
        y_peer = (my_x, 1 - my_y)
        x_peer = (1 - my_x, my_y)

        send_off = my_x * h
        keep_off = (1 - my_x) * h
        own_base = my_y * m
        other_base = (1 - my_y) * m

        def load_s(k):
            c = pltpu.make_async_copy(
                x_hbm.at[pl.ds(send_off + k * rp, rp)],
                f32s.at[k % 2], f32s_sems.at[k % 2])
            c.start()
            return c

        def load_o(k):
            c = pltpu.make_async_copy(
                x_hbm.at[pl.ds(keep_off + k * rp, rp)],
                f32o.at[k % 2], f32o_sems.at[k % 2])
            c.start()
            return c

        s_loads = [None] * K
        o_loads = [None] * K
        for k in range(2):
            s_loads[k] = load_s(k)
            o_loads[k] = load_o(k)

        barrier_sem = pltpu.get_barrier_semaphore()
        for nbr in (y_peer, x_peer):
            pl.semaphore_signal(
                barrier_sem, inc=1, device_id=nbr,
                device_id_type=pl.DeviceIdType.MESH,
            )
        pl.semaphore_wait(barrier_sem, 2)

        y_rdmas = [None] * K
        st_a = [None] * K
        own_stores = [None] * K
        for k in range(K):
            s_loads[k].wait()
            y_send_buf[k, :, :] = f32s[k % 2, :, :].astype(jnp.bfloat16)
            if k + 2 < K:
                s_loads[k + 2] = load_s(k + 2)
            r = pltpu.make_async_remote_copy(
                src_ref=y_send_buf.at[k],
                dst_ref=y_recv_buf.at[k],
                send_sem=y_send_sems.at[k],
                recv_sem=y_recv_sems.at[k],
                device_id=y_peer,
                device_id_type=pl.DeviceIdType.MESH,
            )
            r.start()
            y_rdmas[k] = r
            sa = pltpu.make_async_copy(
                y_send_buf.at[k],
                out_hbm.at[pl.ds(own_base + send_off + k * rp, rp)],
                st_a_sems.at[k])
            sa.start()
            st_a[k] = sa

            o_loads[k].wait()
            if k >= 2:
                own_stores[k - 2].wait()
            own_bf[k % 2, :, :] = f32o[k % 2, :, :].astype(jnp.bfloat16)
            if k + 2 < K:
                o_loads[k + 2] = load_o(k + 2)
            so = pltpu.make_async_copy(
                own_bf.at[k % 2],
                out_hbm.at[pl.ds(own_base + keep_off + k * rp, rp)],
                st_own_sems.at[k % 2])
            so.start()
            own_stores[k] = so

        x_rdmas = [None] * K
        st_y = [None] * K
        for k in range(K):
            y_rdmas[k].wait_recv()
            f = pltpu.make_async_remote_copy(
                src_ref=y_recv_buf.at[k],
                dst_ref=x_recv_buf.at[k],
                send_sem=x_send_sems.at[k],
                recv_sem=x_recv_sems.at[k],
                device_id=x_peer,
                device_id_type=pl.DeviceIdType.MESH,
            )
            f.start()
            x_rdmas[k] = f
            s = pltpu.make_async_copy(
                y_recv_buf.at[k],
                out_hbm.at[pl.ds(other_base + send_off + k * rp, rp)],
                st_y_sems.at[k])
            s.start()
            st_y[k] = s

        st_x = [None] * K
        for k in range(K):
            x_rdmas[k].wait_recv()
            s = pltpu.make_async_copy(
                x_recv_buf.at[k],
                out_hbm.at[pl.ds(other_base + keep_off + k * rp, rp)],
                st_x_sems.at[k])
            s.start()
            st_x[k] = s

        for k in range(K):
            y_rdmas[k].wait_send()
            x_rdmas[k].wait_send()
            st_a[k].wait()
            st_y[k].wait()
            st_x[k].wait()
        own_stores[K - 2].wait()
        own_stores[K - 1].wait()

    return pl.pallas_call(
        body,
        out_shape=jax.ShapeDtypeStruct((2 * m, n), jnp.bfloat16),
        in_specs=[pl.BlockSpec(memory_space=pl.ANY)],
        out_specs=pl.BlockSpec(memory_space=pl.ANY),
        scratch_shapes=[
            pltpu.VMEM((2, rp, n), jnp.float32),
            pltpu.VMEM((2, rp, n), jnp.float32),
            pltpu.VMEM((K, rp, n), jnp.bfloat16),
            pltpu.VMEM((2, rp, n), jnp.bfloat16),
            pltpu.VMEM((K, rp, n), jnp.bfloat16),
            pltpu.VMEM((K, rp, n), jnp.bfloat16),
            pltpu.SemaphoreType.DMA((2,)),
            pltpu.SemaphoreType.DMA((2,)),
            pltpu.SemaphoreType.DMA((2,)),
            pltpu.SemaphoreType.DMA((K,)),
            pltpu.SemaphoreType.DMA((K,)),
            pltpu.SemaphoreType.DMA((K,)),
            pltpu.SemaphoreType.DMA((K,)),
            pltpu.SemaphoreType.DMA((K,)),
            pltpu.SemaphoreType.DMA((K,)),
            pltpu.SemaphoreType.DMA((K,)),
        ],
        compiler_params=pltpu.CompilerParams(
            collective_id=0,
            vmem_limit_bytes=62 * 1024 * 1024,
        ),
    )(x)


# device time: 258497 ns/iter; 1.0113x vs baseline; 1.0113x over previous
import jax
import jax.numpy as jnp
from jax import lax
from jax.experimental import pallas as pl
from jax.experimental.pallas import tpu as pltpu

K = 16


def kernel(x):
    m, n = x.shape
    h = m // 2
    rp = h // K

    def body(x_hbm, out_hbm,
             f32s, f32o, y_send_buf, own_bf, y_recv_buf, x_recv_buf,
             f32s_sems, f32o_sems, st_own_sems,
             y_send_sems, y_recv_sems, x_send_sems, x_recv_sems,
             st_a_sems, st_y_sems, st_x_sems):
        my_x = lax.axis_index("x")
        my_y = lax.axis_index("y")
        y_peer = (my_x, 1 - my_y)
        x_peer = (1 - my_x, my_y)

        send_off = my_x * h
        keep_off = (1 - my_x) * h
        own_base = my_y * m
        other_base = (1 - my_y) * m

        def load_s(k):
            c = pltpu.make_async_copy(
                x_hbm.at[pl.ds(send_off + k * rp, rp)],
                f32s.at[k % 2], f32s_sems.at[k % 2])
            c.start()
            return c

        def load_o(k):
            c = pltpu.make_async_copy(
                x_hbm.at[pl.ds(keep_off + k * rp, rp)],
                f32o.at[k % 2], f32o_sems.at[k % 2])
            c.start()
            return c

        s_loads = [None] * K
        o_loads = [None] * K
        for k in range(2):
            s_loads[k] = load_s(k)
            o_loads[k] = load_o(k)

        barrier_sem = pltpu.get_barrier_semaphore()
        for nbr in (y_peer, x_peer):
            pl.semaphore_signal(
                barrier_sem, inc=1, device_id=nbr,
                device_id_type=pl.DeviceIdType.MESH,
            )
        pl.semaphore_wait(barrier_sem, 2)

        y_rdmas = [None] * K
        st_a = [None] * K
        own_stores = [None] * K
        for k in range(K):
            s_loads[k].wait()
            y_send_buf[k, :, :] = f32s[k % 2, :, :].astype(jnp.bfloat16)
            if k + 2 < K:
                s_loads[k + 2] = load_s(k + 2)
            r = pltpu.make_async_remote_copy(
                src_ref=y_send_buf.at[k],
                dst_ref=y_recv_buf.at[k],
                send_sem=y_send_sems.at[k],
                recv_sem=y_recv_sems.at[k],
                device_id=y_peer,
                device_id_type=pl.DeviceIdType.MESH,
            )
            r.start()
            y_rdmas[k] = r
            sa = pltpu.make_async_copy(
                y_send_buf.at[k],
                out_hbm.at[pl.ds(own_base + send_off + k * rp, rp)],
                st_a_sems.at[k])
            sa.start()
            st_a[k] = sa

            o_loads[k].wait()
            if k >= 2:
                own_stores[k - 2].wait()
            own_bf[k % 2, :, :] = f32o[k % 2, :, :].astype(jnp.bfloat16)
            if k + 2 < K:
                o_loads[k + 2] = load_o(k + 2)
            so = pltpu.make_async_copy(
                own_bf.at[k % 2],
                out_hbm.at[pl.ds(own_base + keep_off + k * rp, rp)],
                st_own_sems.at[k % 2])
            so.start()
            own_stores[k] = so

        x_rdmas = [None] * K
        st_y = [None] * K
        for k in range(K):
            y_rdmas[k].wait_recv()
            f = pltpu.make_async_remote_copy(
                src_ref=y_recv_buf.at[k],
                dst_ref=x_recv_buf.at[k],
                send_sem=x_send_sems.at[k],
                recv_sem=x_recv_sems.at[k],
                device_id=x_peer,
                device_id_type=pl.DeviceIdType.MESH,
            )
            f.start()
            x_rdmas[k] = f
            s = pltpu.make_async_copy(
                y_recv_buf.at[k],
                out_hbm.at[pl.ds(other_base + send_off + k * rp, rp)],
                st_y_sems.at[k])
            s.start()
            st_y[k] = s

        st_x = [None] * K
        for k in range(K):
            x_rdmas[k].wait_recv()
            s = pltpu.make_async_copy(
                x_recv_buf.at[k],
                out_hbm.at[pl.ds(other_base + keep_off + k * rp, rp)],
                st_x_sems.at[k])
            s.start()
            st_x[k] = s

        for k in range(K):
            y_rdmas[k].wait_send()
            x_rdmas[k].wait_send()
            st_a[k].wait()
            st_y[k].wait()
            st_x[k].wait()
        own_stores[K - 2].wait()
        own_stores[K - 1].wait()

    return pl.pallas_call(
        body,
        out_shape=jax.ShapeDtypeStruct((2 * m, n), jnp.bfloat16),
        in_specs=[pl.BlockSpec(memory_space=pl.ANY)],
        out_specs=pl.BlockSpec(memory_space=pl.ANY),
        scratch_shapes=[
            pltpu.VMEM((2, rp, n), jnp.float32),
            pltpu.VMEM((2, rp, n), jnp.float32),
            pltpu.VMEM((K, rp, n), jnp.bfloat16),
            pltpu.VMEM((2, rp, n), jnp.bfloat16),
            pltpu.VMEM((K, rp, n), jnp.bfloat16),
            pltpu.VMEM((K, rp, n), jnp.bfloat16),
            pltpu.SemaphoreType.DMA((2,)),
            pltpu.SemaphoreType.DMA((2,)),
            pltpu.SemaphoreType.DMA((2,)),
            pltpu.SemaphoreType.DMA((K,)),
            pltpu.SemaphoreType.DMA((K,)),
            pltpu.SemaphoreType.DMA((K,)),
            pltpu.SemaphoreType.DMA((K,)),
            pltpu.SemaphoreType.DMA((K,)),
            pltpu.SemaphoreType.DMA((K,)),
            pltpu.SemaphoreType.DMA((K,)),
        ],
        compiler_params=pltpu.CompilerParams(
            collective_id=0,
            vmem_limit_bytes=62 * 1024 * 1024,
        ),
    )(x)


# device time: 258378 ns/iter; 1.0118x vs baseline; 1.0005x over previous
import jax
import jax.numpy as jnp
from jax import lax
from jax.experimental import pallas as pl
from jax.experimental.pallas import tpu as pltpu

K = 16


def kernel(x):
    m, n = x.shape
    h = m // 2
    rp = h // K

    def body(x_hbm, out_hbm,
             f32s, f32o, y_send_buf, own_bf, y_recv_buf, x_recv_buf,
             f32s_sems, f32o_sems, st_own_sems,
             y_send_sems, y_recv_sems, x_send_sems, x_recv_sems,
             st_a_sems, st_y_sems, st_x_sems):
        my_x = lax.axis_index("x")
        my_y = lax.axis_index("y")
        y_peer = (my_x, 1 - my_y)
        x_peer = (1 - my_x, my_y)

        send_off = my_x * h
        keep_off = (1 - my_x) * h
        own_base = my_y * m
        other_base = (1 - my_y) * m

        def load_s(k):
            c = pltpu.make_async_copy(
                x_hbm.at[pl.ds(send_off + k * rp, rp)],
                f32s.at[k % 2], f32s_sems.at[k % 2])
            c.start()
            return c

        def load_o(k):
            c = pltpu.make_async_copy(
                x_hbm.at[pl.ds(keep_off + k * rp, rp)],
                f32o.at[k % 2], f32o_sems.at[k % 2])
            c.start()
            return c

        s_loads = [None] * K
        o_loads = [None] * K
        for k in range(2):
            s_loads[k] = load_s(k)
            o_loads[k] = load_o(k)
        s_loads[0].wait()
        y_send_buf[0, :, :] = f32s[0, :, :].astype(jnp.bfloat16)
        s_loads[2] = load_s(2)

        barrier_sem = pltpu.get_barrier_semaphore()
        for nbr in (y_peer, x_peer):
            pl.semaphore_signal(
                barrier_sem, inc=1, device_id=nbr,
                device_id_type=pl.DeviceIdType.MESH,
            )
        pl.semaphore_wait(barrier_sem, 2)

        y_rdmas = [None] * K
        st_a = [None] * K
        own_stores = [None] * K
        for k in range(K):
            if k > 0:
                s_loads[k].wait()
                y_send_buf[k, :, :] = f32s[k % 2, :, :].astype(jnp.bfloat16)
                if k + 2 < K:
                    s_loads[k + 2] = load_s(k + 2)
            r = pltpu.make_async_remote_copy(
                src_ref=y_send_buf.at[k],
                dst_ref=y_recv_buf.at[k],
                send_sem=y_send_sems.at[k],
                recv_sem=y_recv_sems.at[k],
                device_id=y_peer,
                device_id_type=pl.DeviceIdType.MESH,
            )
            r.start()
            y_rdmas[k] = r
            sa = pltpu.make_async_copy(
                y_send_buf.at[k],
                out_hbm.at[pl.ds(own_base + send_off + k * rp, rp)],
                st_a_sems.at[k])
            sa.start()
            st_a[k] = sa

            o_loads[k].wait()
            if k >= 2:
                own_stores[k - 2].wait()
            own_bf[k % 2, :, :] = f32o[k % 2, :, :].astype(jnp.bfloat16)
            if k + 2 < K:
                o_loads[k + 2] = load_o(k + 2)
            so = pltpu.make_async_copy(
                own_bf.at[k % 2],
                out_hbm.at[pl.ds(own_base + keep_off + k * rp, rp)],
                st_own_sems.at[k % 2])
            so.start()
            own_stores[k] = so

        x_rdmas = [None] * K
        st_y = [None] * K
        for k in range(K):
            y_rdmas[k].wait_recv()
            f = pltpu.make_async_remote_copy(
                src_ref=y_recv_buf.at[k],
                dst_ref=x_recv_buf.at[k],
                send_sem=x_send_sems.at[k],
                recv_sem=x_recv_sems.at[k],
                device_id=x_peer,
                device_id_type=pl.DeviceIdType.MESH,
            )
            f.start()
            x_rdmas[k] = f
            s = pltpu.make_async_copy(
                y_recv_buf.at[k],
                out_hbm.at[pl.ds(other_base + send_off + k * rp, rp)],
                st_y_sems.at[k])
            s.start()
            st_y[k] = s

        st_x = [None] * K
        for k in range(K):
            x_rdmas[k].wait_recv()
            s = pltpu.make_async_copy(
                x_recv_buf.at[k],
                out_hbm.at[pl.ds(other_base + keep_off + k * rp, rp)],
                st_x_sems.at[k])
            s.start()
            st_x[k] = s

        for k in range(K):
            y_rdmas[k].wait_send()
            x_rdmas[k].wait_send()
            st_a[k].wait()
            st_y[k].wait()
            st_x[k].wait()
        own_stores[K - 2].wait()
        own_stores[K - 1].wait()

    return pl.pallas_call(
        body,
        out_shape=jax.ShapeDtypeStruct((2 * m, n), jnp.bfloat16),
        in_specs=[pl.BlockSpec(memory_space=pl.ANY)],
        out_specs=pl.BlockSpec(memory_space=pl.ANY),
        scratch_shapes=[
            pltpu.VMEM((2, rp, n), jnp.float32),
            pltpu.VMEM((2, rp, n), jnp.float32),
            pltpu.VMEM((K, rp, n), jnp.bfloat16),
            pltpu.VMEM((2, rp, n), jnp.bfloat16),
            pltpu.VMEM((K, rp, n), jnp.bfloat16),
            pltpu.VMEM((K, rp, n), jnp.bfloat16),
            pltpu.SemaphoreType.DMA((2,)),
            pltpu.SemaphoreType.DMA((2,)),
            pltpu.SemaphoreType.DMA((2,)),
            pltpu.SemaphoreType.DMA((K,)),
            pltpu.SemaphoreType.DMA((K,)),
            pltpu.SemaphoreType.DMA((K,)),
            pltpu.SemaphoreType.DMA((K,)),
            pltpu.SemaphoreType.DMA((K,)),
            pltpu.SemaphoreType.DMA((K,)),
            pltpu.SemaphoreType.DMA((K,)),
        ],
        compiler_params=pltpu.CompilerParams(
            collective_id=0,
            vmem_limit_bytes=62 * 1024 * 1024,
        ),
    )(x)
